# Initial kernel scaffold; baseline (speedup 1.0000x reference)
import jax
import jax.numpy as jnp
from jax.experimental import pallas as pl


def kernel(nbrs, x_i, scaled_v, W, b):
    raise NotImplementedError("write your pallas kernel here")


# speedup vs baseline: 4.6600x; 4.6600x over previous
import functools

import jax
import jax.numpy as jnp
from jax import lax
from jax.experimental import pallas as pl
from jax.experimental.pallas import tpu as pltpu
from jax.experimental.pallas import tpu_sc as plsc

N = 10000
E = 320000
F = 128
H = 2

NC = 2
NS = 16

G = 128
NG = E // G
GC = 8
SUB = 2
NCH_FULL = NG // GC
NG_PAD = (NCH_FULL + 1) * GC

ROWS_A = 632
ROWS_B = N - 15 * ROWS_A


def _make_sc_scatter():
    mesh = plsc.VectorSubcoreMesh(core_axis_name="c", subcore_axis_name="s")

    @functools.partial(
        pl.kernel,
        out_type=jax.ShapeDtypeStruct((NC, N, F), jnp.float32),
        mesh=mesh,
        scratch_types=[
            pltpu.VMEM((GC, G), jnp.int32),
            pltpu.VMEM((SUB * G, F), jnp.float32),
            pltpu.VMEM_SHARED((N, F), jnp.float32),
        ],
    )
    def sc_scatter(idx_hbm, sv_hbm, zeros_hbm, out_hbm, idx_buf, upd_buf, acc):
        c = lax.axis_index("c")
        s = lax.axis_index("s")
        col = pl.multiple_of(c * F, F)
        r0 = pl.multiple_of(s * ROWS_A, 8)

        @pl.when(s < NS - 1)
        def _():
            pltpu.sync_copy(zeros_hbm, acc.at[pl.ds(r0, ROWS_A), :])

        @pl.when(s == NS - 1)
        def _():
            pltpu.sync_copy(zeros_hbm.at[pl.ds(0, ROWS_B), :],
                            acc.at[pl.ds(15 * ROWS_A, ROWS_B), :])

        plsc.subcore_barrier()

        m0 = jnp.where(s < 8, 20 * s, 160 + 19 * (s - 8))
        nch = jnp.where(s < 8, 20, 19)

        def chunk_body(k, carry):
            gg = pl.multiple_of((m0 + k) * GC, 8)
            pltpu.sync_copy(idx_hbm.at[pl.ds(gg, GC), :], idx_buf)
            for t in range(GC // SUB):
                e0 = pl.multiple_of((gg + t * SUB) * G, 8)
                pltpu.sync_copy(sv_hbm.at[pl.ds(e0, SUB * G), pl.ds(col, F)],
                                upd_buf)
                for j in range(SUB):
                    pltpu.sync_copy(
                        upd_buf.at[pl.ds(j * G, G), :],
                        acc.at[idx_buf.at[t * SUB + j]],
                        add=True,
                    )
            return carry

        lax.fori_loop(0, nch, chunk_body, 0)

        @pl.when(s == NS - 1)
        def _():
            gg = NCH_FULL * GC
            pltpu.sync_copy(idx_hbm.at[pl.ds(gg, GC), :], idx_buf)
            for t in range((NG - NCH_FULL * GC) // SUB):
                e0 = (gg + t * SUB) * G
                pltpu.sync_copy(sv_hbm.at[pl.ds(e0, SUB * G), pl.ds(col, F)],
                                upd_buf)
                for j in range(SUB):
                    pltpu.sync_copy(
                        upd_buf.at[pl.ds(j * G, G), :],
                        acc.at[idx_buf.at[t * SUB + j]],
                        add=True,
                    )

        plsc.subcore_barrier()

        @pl.when(s < NS - 1)
        def _():
            pltpu.sync_copy(acc.at[pl.ds(r0, ROWS_A), :],
                            out_hbm.at[c, pl.ds(r0, ROWS_A), :])

        @pl.when(s == NS - 1)
        def _():
            pltpu.sync_copy(acc.at[pl.ds(15 * ROWS_A, ROWS_B), :],
                            out_hbm.at[c, pl.ds(15 * ROWS_A, ROWS_B), :])

    return sc_scatter


_sc_scatter = _make_sc_scatter()


def _mm_body(xp_ref, x_ref, w_ref, b_ref, o_ref):
    o_ref[...] = (
        x_ref[...]
        + b_ref[...]
        + jnp.dot(xp_ref[0], w_ref[0], preferred_element_type=jnp.float32)
        + jnp.dot(xp_ref[1], w_ref[1], preferred_element_type=jnp.float32)
    )


def _tc_dense(xp, x_i, W2, b2):
    BN = 2000
    grid = (N // BN,)
    return pl.pallas_call(
        _mm_body,
        grid=grid,
        in_specs=[
            pl.BlockSpec((H, BN, F), lambda i: (0, i, 0)),
            pl.BlockSpec((BN, F), lambda i: (i, 0)),
            pl.BlockSpec((H, F, F), lambda i: (0, 0, 0)),
            pl.BlockSpec((1, F), lambda i: (0, 0)),
        ],
        out_specs=pl.BlockSpec((BN, F), lambda i: (i, 0)),
        out_shape=jax.ShapeDtypeStruct((N, F), jnp.float32),
    )(xp, x_i, W2, b2)


def kernel(nbrs, x_i, scaled_v, W, b):
    idx2d = nbrs[:, 0].astype(jnp.int32).reshape(NG, G)
    idx2d = jnp.pad(idx2d, ((0, NG_PAD - NG), (0, 0)))
    zeros = jnp.zeros((ROWS_A, F), jnp.float32)
    xp = _sc_scatter(idx2d, scaled_v, zeros)
    return _tc_dense(xp, x_i, W.reshape(H, F, F), b.reshape(1, F))


# speedup vs baseline: 6.6165x; 1.4198x over previous
import functools

import jax
import jax.numpy as jnp
from jax import lax
from jax.experimental import pallas as pl
from jax.experimental.pallas import tpu as pltpu
from jax.experimental.pallas import tpu_sc as plsc

N = 10000
E = 320000
F = 128
H = 2

NC = 2
NS = 16

G = 128
NG = E // G
GC = 8
NCH_FULL = NG // GC
NG_PAD = (NCH_FULL + 1) * GC

ROWS_A = 632
ROWS_B = N - 15 * ROWS_A


def _make_sc_scatter():
    mesh = plsc.VectorSubcoreMesh(core_axis_name="c", subcore_axis_name="s")

    @functools.partial(
        pl.kernel,
        out_type=jax.ShapeDtypeStruct((NC, N, F), jnp.float32),
        mesh=mesh,
        scratch_types=[
            pltpu.VMEM((GC, G), jnp.int32),
            pltpu.VMEM((G, F), jnp.float32),
            pltpu.VMEM((G, F), jnp.float32),
            pltpu.VMEM_SHARED((N, F), jnp.float32),
            pltpu.SemaphoreType.DMA,
            pltpu.SemaphoreType.DMA,
        ],
    )
    def sc_scatter(idx_hbm, sv_hbm, zeros_hbm, out_hbm,
                   idx_buf, ubuf0, ubuf1, acc, sem0, sem1):
        c = lax.axis_index("c")
        s = lax.axis_index("s")
        col = pl.multiple_of(c * F, F)
        r0 = pl.multiple_of(s * ROWS_A, 8)

        ubufs = (ubuf0, ubuf1)
        sems = (sem0, sem1)

        m0 = jnp.where(s < 8, 20 * s, 160 + 19 * (s - 8))
        nch = jnp.where(s < 8, 20, 19)
        g0 = m0 * GC
        ng_main = nch * GC

        def src_slice(flat):
            e0 = pl.multiple_of((g0 + flat) * G, 8)
            return sv_hbm.at[pl.ds(e0, G), pl.ds(col, F)]

        pltpu.async_copy(src_slice(0), ubuf0, sem0)
        pltpu.async_copy(src_slice(1), ubuf1, sem1)

        @pl.when(s < NS - 1)
        def _():
            pltpu.sync_copy(zeros_hbm, acc.at[pl.ds(r0, ROWS_A), :])

        @pl.when(s == NS - 1)
        def _():
            pltpu.sync_copy(zeros_hbm.at[pl.ds(0, ROWS_B), :],
                            acc.at[pl.ds(15 * ROWS_A, ROWS_B), :])

        plsc.subcore_barrier()

        def chunk_body(k, carry):
            gg = pl.multiple_of((m0 + k) * GC, 8)
            pltpu.sync_copy(idx_hbm.at[pl.ds(gg, GC), :], idx_buf)
            for j in range(GC):
                b = j % 2
                flat = k * GC + j
                pltpu.make_async_copy(src_slice(flat), ubufs[b], sems[b]).wait()
                pltpu.sync_copy(ubufs[b], acc.at[idx_buf.at[j]], add=True)

                @pl.when(flat + 2 < ng_main)
                def _():
                    pltpu.async_copy(src_slice(flat + 2), ubufs[b], sems[b])
            return carry

        lax.fori_loop(0, nch, chunk_body, 0)

        @pl.when(s == NS - 1)
        def _():
            gg = NCH_FULL * GC
            pltpu.sync_copy(idx_hbm.at[pl.ds(gg, GC), :], idx_buf)
            for j in range(NG - NCH_FULL * GC):
                e0 = (gg + j) * G
                pltpu.sync_copy(sv_hbm.at[pl.ds(e0, G), pl.ds(col, F)], ubuf0)
                pltpu.sync_copy(ubuf0, acc.at[idx_buf.at[j]], add=True)

        plsc.subcore_barrier()

        @pl.when(s < NS - 1)
        def _():
            pltpu.sync_copy(acc.at[pl.ds(r0, ROWS_A), :],
                            out_hbm.at[c, pl.ds(r0, ROWS_A), :])

        @pl.when(s == NS - 1)
        def _():
            pltpu.sync_copy(acc.at[pl.ds(15 * ROWS_A, ROWS_B), :],
                            out_hbm.at[c, pl.ds(15 * ROWS_A, ROWS_B), :])

    return sc_scatter


_sc_scatter = _make_sc_scatter()


def _mm_body(xp_ref, x_ref, w_ref, b_ref, o_ref):
    o_ref[...] = (
        x_ref[...]
        + b_ref[...]
        + jnp.dot(xp_ref[0], w_ref[0], preferred_element_type=jnp.float32)
        + jnp.dot(xp_ref[1], w_ref[1], preferred_element_type=jnp.float32)
    )


def _tc_dense(xp, x_i, W2, b2):
    BN = 2000
    grid = (N // BN,)
    return pl.pallas_call(
        _mm_body,
        grid=grid,
        in_specs=[
            pl.BlockSpec((H, BN, F), lambda i: (0, i, 0)),
            pl.BlockSpec((BN, F), lambda i: (i, 0)),
            pl.BlockSpec((H, F, F), lambda i: (0, 0, 0)),
            pl.BlockSpec((1, F), lambda i: (0, 0)),
        ],
        out_specs=pl.BlockSpec((BN, F), lambda i: (i, 0)),
        out_shape=jax.ShapeDtypeStruct((N, F), jnp.float32),
    )(xp, x_i, W2, b2)


def kernel(nbrs, x_i, scaled_v, W, b):
    idx2d = nbrs[:, 0].astype(jnp.int32).reshape(NG, G)
    idx2d = jnp.pad(idx2d, ((0, NG_PAD - NG), (0, 0)))
    zeros = jnp.zeros((ROWS_A, F), jnp.float32)
    xp = _sc_scatter(idx2d, scaled_v, zeros)
    return _tc_dense(xp, x_i, W.reshape(H, F, F), b.reshape(1, F))
